# baseline (speedup 1.0000x reference)
import functools

import jax
import jax.numpy as jnp
from jax import lax
from jax.experimental import pallas as pl
from jax.experimental.pallas import tpu as pltpu
from jax.experimental.pallas import tpu_sc as plsc


def _tc_pre_body(x_ref, w_ref, w1_ref, w2_ref, bsum_ref, h2_ref, a1_ref, a2_ref):
    n = x_ref.shape[2]
    dh = h2_ref.shape[1]
    xb = x_ref[0]
    h = lax.dot_general(
        xb, w_ref[...], (((0,), (1,)), ((), ())),
        preferred_element_type=jnp.float32,
    )
    ii = lax.broadcasted_iota(jnp.int32, (dh, dh), 0)
    jj = lax.broadcasted_iota(jnp.int32, (dh, dh), 1)
    srcc = (jj // 32) * 32 + (jj % 2) * 16 + (jj % 32) // 2
    perm = (ii == srcc).astype(jnp.float32)
    for half in range(2):
        blk = lax.dot_general(
            h[:, half * dh:(half + 1) * dh], perm,
            (((1,), (0,)), ((), ())), preferred_element_type=jnp.float32)
        h2_ref[pl.ds(half * n, n), :] = blk.astype(jnp.bfloat16)
    a1_ref[...] = jnp.sum(h * w1_ref[0][None, :], axis=1) + bsum_ref[0, 0]
    a2_ref[...] = jnp.sum(h * w2_ref[0][None, :], axis=1)


@functools.lru_cache(maxsize=None)
def _tc_pre(n, d_in, d_out):
    return pl.pallas_call(
        _tc_pre_body,
        out_shape=[
            jax.ShapeDtypeStruct((2 * n, d_out // 2), jnp.bfloat16),
            jax.ShapeDtypeStruct((n,), jnp.float32),
            jax.ShapeDtypeStruct((n,), jnp.float32),
        ],
    )


@functools.lru_cache(maxsize=None)
def _sc_main(n, e, d):
    info = plsc.get_sparse_core_info()
    nc, ns, lanes = info.num_cores, info.num_subcores, info.num_lanes
    dh = d // nc
    ew = e // ns
    bsz = 80
    nrow = ew // bsz
    nbuf = 5
    seg = 25
    nseg = nrow // seg
    npad = -(-n // (ns * 128)) * (ns * 128)
    rt = npad // ns
    zr = 64
    assert e % ns == 0 and ew % bsz == 0 and bsz % lanes == 0
    assert nrow % seg == 0 and nseg % 2 == 0 and seg % nbuf == 0
    assert n % lanes == 0 and rt % zr == 0 and dh % lanes == 0

    mesh = plsc.VectorSubcoreMesh(core_axis_name="c", subcore_axis_name="s")

    @functools.partial(
        pl.kernel,
        mesh=mesh,
        compiler_params=pltpu.CompilerParams(
            needs_layout_passes=False, use_tc_tiling_on_sc=False),
        out_type=[
            jax.ShapeDtypeStruct((nc, npad, dh), jnp.float32),
            jax.ShapeDtypeStruct((ns * n,), jnp.float32),
        ],
        scratch_types=[
            [pltpu.VMEM((seg, bsz), jnp.int32) for _ in range(2)],
            [pltpu.VMEM((seg, bsz), jnp.int32) for _ in range(2)],
            [pltpu.VMEM((seg, bsz), jnp.float32) for _ in range(2)],
            pltpu.VMEM((n,), jnp.float32),
            pltpu.VMEM((n,), jnp.float32),
            pltpu.VMEM((n,), jnp.float32),
            pltpu.VMEM((zr, dh), jnp.float32),
            [pltpu.VMEM((bsz, dh), jnp.bfloat16) for _ in range(nbuf)],
            [pltpu.VMEM((bsz, dh), jnp.float32) for _ in range(nbuf)],
            [pltpu.SemaphoreType.DMA for _ in range(2)],
            [pltpu.SemaphoreType.DMA for _ in range(nbuf)],
            [pltpu.SemaphoreType.DMA for _ in range(nbuf)],
            pltpu.VMEM_SHARED((npad, dh), jnp.float32),
        ],
    )
    def sc(src_hbm, dst_hbm, a1_hbm, a2_hbm, h_hbm, u_out, s_out,
           srcs, dsts, es, a1_v, a2_v, s_v, zbuf, hbufs, obufs, stsems,
           gsems, ssems, u_sh):
        cid = lax.axis_index("c")
        sid = lax.axis_index("s")
        zero16 = jnp.zeros((lanes,), jnp.float32)

        def zero_zbuf(i, carry):
            for c in range(dh // lanes):
                zbuf[i, pl.ds(c * lanes, lanes)] = zero16
            return carry

        lax.fori_loop(0, zr, zero_zbuf, 0)

        def zero_s(i, carry):
            s_v[pl.ds(i * lanes, lanes)] = zero16
            return carry

        lax.fori_loop(0, n // lanes, zero_s, 0)

        row0 = sid * rt
        for k in range(rt // zr):
            pltpu.sync_copy(zbuf, u_sh.at[pl.ds(row0 + k * zr, zr)])
        plsc.subcore_barrier()

        pltpu.sync_copy(a1_hbm, a1_v)
        pltpu.sync_copy(a2_hbm, a2_v)

        hrow_base = cid * n
        brow0 = sid * nrow

        def fire_stage(p, g):
            r = pl.ds(brow0 + g * seg, seg)
            pltpu.async_copy(src_hbm.at[r], srcs[p], stsems[p])
            pltpu.async_copy(dst_hbm.at[r], dsts[p], stsems[p])

        def wait_stage(p, g):
            r = pl.ds(brow0 + g * seg, seg)
            pltpu.make_async_copy(src_hbm.at[r], srcs[p], stsems[p]).wait()
            pltpu.make_async_copy(dst_hbm.at[r], dsts[p], stsems[p]).wait()

        def scalar_row(p, j):
            for v in range(bsz // lanes):
                sl = pl.ds(v * lanes, lanes)
                s16 = srcs[p][j, sl]
                d16 = dsts[p][j, sl]
                av = (plsc.load_gather(a1_v, [s16])
                      + plsc.load_gather(a2_v, [d16]))
                av = jnp.where(av > 0, av, 0.01 * av)
                ev = jnp.exp(av)
                plsc.addupdate_scatter(s_v, [s16], ev)
                es[p][j, sl] = ev
                dsts[p][j, sl] = d16 + hrow_base

        def scale(p, b, j):
            def sub(v, carry):
                ev = es[p][j, pl.ds(v * lanes, lanes)]
                himask = jnp.full((lanes,), -65536, jnp.int32)
                for r in range(lanes):
                    er = ev[r]
                    row = v * lanes + r
                    for c in range(dh // (2 * lanes)):
                        hb = hbufs[b][row, pl.ds(c * 2 * lanes, 2 * lanes)]
                        iv = plsc.bitcast(hb, jnp.int32)
                        lo = plsc.bitcast(iv << 16, jnp.float32)
                        hi = plsc.bitcast(iv & himask, jnp.float32)
                        obufs[b][row, pl.ds(c * 2 * lanes, lanes)] = lo * er
                        obufs[b][row, pl.ds(c * 2 * lanes + lanes, lanes)] = (
                            hi * er)
                return carry
            lax.fori_loop(0, bsz // lanes, sub, 0)

        def fire_gather(p, b, j):
            pltpu.async_copy(h_hbm.at[dsts[p].at[j]], hbufs[b], gsems[b])

        def fire_scatter(p, b, j):
            pltpu.async_copy(obufs[b], u_sh.at[srcs[p].at[j]], ssems[b],
                             add=True)

        def wait_gather(p, b, j):
            pltpu.make_async_copy(h_hbm.at[dsts[p].at[j]], hbufs[b],
                                  gsems[b]).wait()

        def wait_scatter(p, b, j):
            pltpu.make_async_copy(obufs[b], u_sh.at[srcs[p].at[j]],
                                  ssems[b]).wait()

        look = 2
        def heavy_pass(p):
            scalar_row(p, 0)
            scalar_row(p, 1)
            fire_gather(p, 0, 0)
            fire_gather(p, 1, 1)

            def ring(k, carry):
                for b in range(nbuf):
                    j = k * nbuf + b
                    bn = (b + look) % nbuf
                    if b < nbuf - look:
                        @pl.when(k > 0)
                        def _():
                            wait_scatter(p, bn, j + look - nbuf)
                        scalar_row(p, j + look)
                        fire_gather(p, bn, j + look)
                    else:
                        wait_scatter(p, bn, j + look - nbuf)

                        @pl.when(k < seg // nbuf - 1)
                        def _():
                            scalar_row(p, j + look)
                            fire_gather(p, bn, j + look)
                    wait_gather(p, b, j)
                    scale(p, b, j)
                    fire_scatter(p, b, j)
                return carry

            lax.fori_loop(0, seg // nbuf, ring, 0)
            for b in range(look, nbuf):
                wait_scatter(p, b, seg - nbuf + b)

        fire_stage(0, 0)

        def segpair(k, carry):
            g0 = 2 * k
            fire_stage(1, g0 + 1)
            wait_stage(0, g0)
            heavy_pass(0)

            @pl.when(k < nseg // 2 - 1)
            def _():
                fire_stage(0, g0 + 2)

            wait_stage(1, g0 + 1)
            heavy_pass(1)
            return carry

        lax.fori_loop(0, nseg // 2, segpair, 0)
        plsc.subcore_barrier()

        for k in range(rt // zr):
            pltpu.sync_copy(u_sh.at[pl.ds(row0 + k * zr, zr)],
                            u_out.at[cid, pl.ds(row0 + k * zr, zr)])

        @pl.when(cid == 0)
        def _():
            pltpu.sync_copy(s_v, s_out.at[pl.ds(sid * n, n)])

    return sc


def _tc_post_body(u_ref, s_ref, bias_ref, o_ref):
    n = o_ref.shape[2]
    acc = jnp.concatenate([u_ref[0, :n], u_ref[1, :n]], axis=1)
    s = jnp.sum(s_ref[...], axis=0)
    den = jnp.where(s > 0, s, 1.0)
    r = acc / den[:, None] + bias_ref[...][None, :]
    r = jnp.where(r > 0, r, jnp.exp(jnp.minimum(r, 0.0)) - 1.0)
    o_ref[...] = jnp.transpose(r)[None]


@functools.lru_cache(maxsize=None)
def _tc_post(n, d):
    return pl.pallas_call(
        _tc_post_body,
        out_shape=jax.ShapeDtypeStruct((1, d, n), jnp.float32),
    )


def kernel(x, edge_index, W, w1, b1, w2, b2, bias):
    _, d_in, n = x.shape
    d_out = W.shape[0]
    e = edge_index.shape[1]
    bsum = jnp.reshape(b1 + b2, (1, 1))
    h2, a1, a2 = _tc_pre(n, d_in, d_out)(x, W, w1, w2, bsum)
    src2 = jnp.reshape(edge_index[0], (-1, 80))
    dst2 = jnp.reshape(edge_index[1], (-1, 80))
    u, s = _sc_main(n, e, d_out)(src2, dst2, a1, a2, h2)
    return _tc_post(n, d_out)(u, jnp.reshape(s, (-1, n)), bias)


# speedup vs baseline: 1.5791x; 1.5791x over previous
import functools

import jax
import jax.numpy as jnp
from jax import lax
from jax.experimental import pallas as pl
from jax.experimental.pallas import tpu as pltpu
from jax.experimental.pallas import tpu_sc as plsc


def _tc_pre_body(x_ref, w_ref, w1_ref, w2_ref, bsum_ref, h2_ref, a1_ref, a2_ref):
    n = x_ref.shape[2]
    dh = h2_ref.shape[1]
    xb = x_ref[0]
    h = lax.dot_general(
        xb, w_ref[...], (((0,), (1,)), ((), ())),
        preferred_element_type=jnp.float32,
    )
    h2_ref[pl.ds(0, n), :] = h[:, :dh]
    h2_ref[pl.ds(n, n), :] = h[:, dh:]
    a1_ref[...] = jnp.sum(h * w1_ref[0][None, :], axis=1) + bsum_ref[0, 0]
    a2_ref[...] = jnp.sum(h * w2_ref[0][None, :], axis=1)


@functools.lru_cache(maxsize=None)
def _tc_pre(n, d_in, d_out):
    return pl.pallas_call(
        _tc_pre_body,
        out_shape=[
            jax.ShapeDtypeStruct((2 * n, d_out // 2), jnp.float32),
            jax.ShapeDtypeStruct((n,), jnp.float32),
            jax.ShapeDtypeStruct((n,), jnp.float32),
        ],
    )


@functools.lru_cache(maxsize=None)
def _sc_main(n, e, d):
    info = plsc.get_sparse_core_info()
    nc, ns, lanes = info.num_cores, info.num_subcores, info.num_lanes
    dh = d // nc
    ew = e // ns
    bsz = 80
    nrow = ew // bsz
    nbuf = 5
    seg = 25
    nseg = nrow // seg
    npad = -(-n // (ns * 128)) * (ns * 128)
    rt = npad // ns
    zr = 64
    assert e % ns == 0 and ew % bsz == 0 and bsz % lanes == 0
    assert nrow % seg == 0 and nseg % 2 == 0 and seg % nbuf == 0
    assert n % lanes == 0 and rt % zr == 0 and dh % lanes == 0

    mesh = plsc.VectorSubcoreMesh(core_axis_name="c", subcore_axis_name="s")

    @functools.partial(
        pl.kernel,
        mesh=mesh,
        compiler_params=pltpu.CompilerParams(
            needs_layout_passes=False, use_tc_tiling_on_sc=False),
        out_type=[
            jax.ShapeDtypeStruct((nc, npad, dh), jnp.float32),
            jax.ShapeDtypeStruct((ns * n,), jnp.float32),
        ],
        scratch_types=[
            [pltpu.VMEM((seg, bsz), jnp.int32) for _ in range(2)],
            [pltpu.VMEM((seg, bsz), jnp.int32) for _ in range(2)],
            [pltpu.VMEM((seg, bsz), jnp.float32) for _ in range(2)],
            pltpu.VMEM((n,), jnp.float32),
            pltpu.VMEM((n,), jnp.float32),
            pltpu.VMEM((n,), jnp.float32),
            pltpu.VMEM((zr, dh), jnp.float32),
            [pltpu.VMEM((bsz, dh), jnp.float32) for _ in range(nbuf)],
            [pltpu.SemaphoreType.DMA for _ in range(2)],
            [pltpu.SemaphoreType.DMA for _ in range(nbuf)],
            [pltpu.SemaphoreType.DMA for _ in range(nbuf)],
            pltpu.VMEM_SHARED((npad, dh), jnp.float32),
        ],
    )
    def sc(src_hbm, dst_hbm, a1_hbm, a2_hbm, h_hbm, u_out, s_out,
           srcs, dsts, es, a1_v, a2_v, s_v, zbuf, hbufs, stsems, gsems,
           ssems, u_sh):
        cid = lax.axis_index("c")
        sid = lax.axis_index("s")
        zero16 = jnp.zeros((lanes,), jnp.float32)

        def zero_zbuf(i, carry):
            for c in range(dh // lanes):
                zbuf[i, pl.ds(c * lanes, lanes)] = zero16
            return carry

        lax.fori_loop(0, zr, zero_zbuf, 0)

        def zero_s(i, carry):
            s_v[pl.ds(i * lanes, lanes)] = zero16
            return carry

        lax.fori_loop(0, n // lanes, zero_s, 0)

        row0 = sid * rt
        for k in range(rt // zr):
            pltpu.sync_copy(zbuf, u_sh.at[pl.ds(row0 + k * zr, zr)])
        plsc.subcore_barrier()

        pltpu.sync_copy(a1_hbm, a1_v)
        pltpu.sync_copy(a2_hbm, a2_v)

        hrow_base = cid * n
        brow0 = sid * nrow

        def fire_stage(p, g):
            r = pl.ds(brow0 + g * seg, seg)
            pltpu.async_copy(src_hbm.at[r], srcs[p], stsems[p])
            pltpu.async_copy(dst_hbm.at[r], dsts[p], stsems[p])

        def wait_stage(p, g):
            r = pl.ds(brow0 + g * seg, seg)
            pltpu.make_async_copy(src_hbm.at[r], srcs[p], stsems[p]).wait()
            pltpu.make_async_copy(dst_hbm.at[r], dsts[p], stsems[p]).wait()

        def scalar_row(p, j):
            for v in range(bsz // lanes):
                sl = pl.ds(v * lanes, lanes)
                s16 = srcs[p][j, sl]
                d16 = dsts[p][j, sl]
                av = (plsc.load_gather(a1_v, [s16])
                      + plsc.load_gather(a2_v, [d16]))
                av = jnp.where(av > 0, av, 0.01 * av)
                ev = jnp.exp(av)
                plsc.addupdate_scatter(s_v, [s16], ev)
                es[p][j, sl] = ev
                dsts[p][j, sl] = d16 + hrow_base

        def scale(p, b, j):
            def sub(v, carry):
                ev = es[p][j, pl.ds(v * lanes, lanes)]
                for r in range(lanes):
                    er = ev[r]
                    row = v * lanes + r
                    for c in range(dh // lanes):
                        sl = pl.ds(c * lanes, lanes)
                        hbufs[b][row, sl] = hbufs[b][row, sl] * er
                return carry
            lax.fori_loop(0, bsz // lanes, sub, 0)

        def fire_gather(p, b, j):
            pltpu.async_copy(h_hbm.at[dsts[p].at[j]], hbufs[b], gsems[b])

        def fire_scatter(p, b, j):
            pltpu.async_copy(hbufs[b], u_sh.at[srcs[p].at[j]], ssems[b],
                             add=True)

        def wait_gather(p, b, j):
            pltpu.make_async_copy(h_hbm.at[dsts[p].at[j]], hbufs[b],
                                  gsems[b]).wait()

        def wait_scatter(p, b, j):
            pltpu.make_async_copy(hbufs[b], u_sh.at[srcs[p].at[j]],
                                  ssems[b]).wait()

        look = 3
        def heavy_pass(p):
            for i in range(look):
                scalar_row(p, i)
                fire_gather(p, i, i)

            def ring(k, carry):
                for b in range(nbuf):
                    j = k * nbuf + b
                    bn = (b + look) % nbuf
                    if b < nbuf - look:
                        @pl.when(k > 0)
                        def _():
                            wait_scatter(p, bn, j + look - nbuf)
                        scalar_row(p, j + look)
                        fire_gather(p, bn, j + look)
                    else:
                        wait_scatter(p, bn, j + look - nbuf)

                        @pl.when(k < seg // nbuf - 1)
                        def _():
                            scalar_row(p, j + look)
                            fire_gather(p, bn, j + look)
                    wait_gather(p, b, j)
                    scale(p, b, j)
                    fire_scatter(p, b, j)
                return carry

            lax.fori_loop(0, seg // nbuf, ring, 0)
            for b in range(look, nbuf):
                wait_scatter(p, b, seg - nbuf + b)

        fire_stage(0, 0)

        def segpair(k, carry):
            g0 = 2 * k
            fire_stage(1, g0 + 1)
            wait_stage(0, g0)
            heavy_pass(0)

            @pl.when(k < nseg // 2 - 1)
            def _():
                fire_stage(0, g0 + 2)

            wait_stage(1, g0 + 1)
            heavy_pass(1)
            return carry

        lax.fori_loop(0, nseg // 2, segpair, 0)
        plsc.subcore_barrier()

        for k in range(rt // zr):
            pltpu.sync_copy(u_sh.at[pl.ds(row0 + k * zr, zr)],
                            u_out.at[cid, pl.ds(row0 + k * zr, zr)])

        @pl.when(cid == 0)
        def _():
            pltpu.sync_copy(s_v, s_out.at[pl.ds(sid * n, n)])

    return sc


def _tc_post_body(u_ref, s_ref, bias_ref, o_ref):
    n = o_ref.shape[2]
    acc = jnp.concatenate([u_ref[0, :n], u_ref[1, :n]], axis=1)
    s = jnp.sum(s_ref[...], axis=0)
    den = jnp.where(s > 0, s, 1.0)
    r = acc / den[:, None] + bias_ref[...][None, :]
    r = jnp.where(r > 0, r, jnp.exp(jnp.minimum(r, 0.0)) - 1.0)
    o_ref[...] = jnp.transpose(r)[None]


@functools.lru_cache(maxsize=None)
def _tc_post(n, d):
    return pl.pallas_call(
        _tc_post_body,
        out_shape=jax.ShapeDtypeStruct((1, d, n), jnp.float32),
    )


def kernel(x, edge_index, W, w1, b1, w2, b2, bias):
    _, d_in, n = x.shape
    d_out = W.shape[0]
    e = edge_index.shape[1]
    bsum = jnp.reshape(b1 + b2, (1, 1))
    h2, a1, a2 = _tc_pre(n, d_in, d_out)(x, W, w1, w2, bsum)
    src2 = jnp.reshape(edge_index[0], (-1, 80))
    dst2 = jnp.reshape(edge_index[1], (-1, 80))
    u, s = _sc_main(n, e, d_out)(src2, dst2, a1, a2, h2)
    return _tc_post(n, d_out)(u, jnp.reshape(s, (-1, n)), bias)


# speedup vs baseline: 1.5794x; 1.0002x over previous
import functools

import jax
import jax.numpy as jnp
from jax import lax
from jax.experimental import pallas as pl
from jax.experimental.pallas import tpu as pltpu
from jax.experimental.pallas import tpu_sc as plsc


def _tc_pre_body(x_ref, w_ref, w1_ref, w2_ref, bsum_ref, h2_ref, a1_ref, a2_ref):
    n = x_ref.shape[2]
    dh = h2_ref.shape[1]
    xb = x_ref[0]
    h = lax.dot_general(
        xb, w_ref[...], (((0,), (1,)), ((), ())),
        preferred_element_type=jnp.float32,
    )
    h2_ref[pl.ds(0, n), :] = h[:, :dh]
    h2_ref[pl.ds(n, n), :] = h[:, dh:]
    a1_ref[...] = jnp.sum(h * w1_ref[0][None, :], axis=1) + bsum_ref[0, 0]
    a2_ref[...] = jnp.sum(h * w2_ref[0][None, :], axis=1)


@functools.lru_cache(maxsize=None)
def _tc_pre(n, d_in, d_out):
    return pl.pallas_call(
        _tc_pre_body,
        out_shape=[
            jax.ShapeDtypeStruct((2 * n, d_out // 2), jnp.float32),
            jax.ShapeDtypeStruct((n,), jnp.float32),
            jax.ShapeDtypeStruct((n,), jnp.float32),
        ],
    )


@functools.lru_cache(maxsize=None)
def _sc_main(n, e, d):
    info = plsc.get_sparse_core_info()
    nc, ns, lanes = info.num_cores, info.num_subcores, info.num_lanes
    dh = d // nc
    ew = e // ns
    bsz = 80
    nrow = ew // bsz
    nbuf = 5
    seg = 25
    nseg = nrow // seg
    npad = -(-n // (ns * 128)) * (ns * 128)
    rt = npad // ns
    zr = 64
    assert e % ns == 0 and ew % bsz == 0 and bsz % lanes == 0
    assert nrow % seg == 0 and nseg % 2 == 0 and seg % nbuf == 0
    assert n % lanes == 0 and rt % zr == 0 and dh % lanes == 0

    mesh = plsc.VectorSubcoreMesh(core_axis_name="c", subcore_axis_name="s")

    @functools.partial(
        pl.kernel,
        mesh=mesh,
        compiler_params=pltpu.CompilerParams(
            needs_layout_passes=False, use_tc_tiling_on_sc=False),
        out_type=[
            jax.ShapeDtypeStruct((nc, npad, dh), jnp.float32),
            jax.ShapeDtypeStruct((ns * n,), jnp.float32),
        ],
        scratch_types=[
            [pltpu.VMEM((seg * bsz,), jnp.int32) for _ in range(2)],
            [pltpu.VMEM((seg * bsz,), jnp.int32) for _ in range(2)],
            [pltpu.VMEM((seg * bsz,), jnp.float32) for _ in range(2)],
            pltpu.VMEM((n,), jnp.float32),
            pltpu.VMEM((n,), jnp.float32),
            pltpu.VMEM((n,), jnp.float32),
            pltpu.VMEM((zr, dh), jnp.float32),
            [pltpu.VMEM((bsz, dh), jnp.float32) for _ in range(nbuf)],
            [pltpu.SemaphoreType.DMA for _ in range(2)],
            [pltpu.SemaphoreType.DMA for _ in range(nbuf)],
            [pltpu.SemaphoreType.DMA for _ in range(nbuf)],
            pltpu.VMEM_SHARED((npad, dh), jnp.float32),
        ],
    )
    def sc(src_hbm, dst_hbm, a1_hbm, a2_hbm, h_hbm, u_out, s_out,
           srcs, dsts, es, a1_v, a2_v, s_v, zbuf, hbufs, stsems, gsems,
           ssems, u_sh):
        cid = lax.axis_index("c")
        sid = lax.axis_index("s")
        zero16 = jnp.zeros((lanes,), jnp.float32)

        def zero_zbuf(i, carry):
            for c in range(dh // lanes):
                zbuf[i, pl.ds(c * lanes, lanes)] = zero16
            return carry

        lax.fori_loop(0, zr, zero_zbuf, 0)

        def zero_s(i, carry):
            s_v[pl.ds(i * lanes, lanes)] = zero16
            return carry

        lax.fori_loop(0, n // lanes, zero_s, 0)

        row0 = sid * rt
        for k in range(rt // zr):
            pltpu.sync_copy(zbuf, u_sh.at[pl.ds(row0 + k * zr, zr)])
        plsc.subcore_barrier()

        pltpu.sync_copy(a1_hbm, a1_v)
        pltpu.sync_copy(a2_hbm, a2_v)

        hrow_base = cid * n
        ebase = sid * ew

        def fire_stage(p, g):
            r = pl.ds(ebase + g * seg * bsz, seg * bsz)
            pltpu.async_copy(src_hbm.at[r], srcs[p], stsems[p])
            pltpu.async_copy(dst_hbm.at[r], dsts[p], stsems[p])

        def wait_stage(p, g):
            r = pl.ds(ebase + g * seg * bsz, seg * bsz)
            pltpu.make_async_copy(src_hbm.at[r], srcs[p], stsems[p]).wait()
            pltpu.make_async_copy(dst_hbm.at[r], dsts[p], stsems[p]).wait()

        def scalar_row(p, j):
            for v in range(bsz // lanes):
                sl = pl.ds(j * bsz + v * lanes, lanes)
                s16 = srcs[p][sl]
                d16 = dsts[p][sl]
                av = (plsc.load_gather(a1_v, [s16])
                      + plsc.load_gather(a2_v, [d16]))
                av = jnp.where(av > 0, av, 0.01 * av)
                ev = jnp.exp(av)
                plsc.addupdate_scatter(s_v, [s16], ev)
                es[p][sl] = ev
                dsts[p][sl] = d16 + hrow_base

        def scale(p, b, j):
            def sub(v, carry):
                ev = es[p][pl.ds(j * bsz + v * lanes, lanes)]
                for r in range(lanes):
                    er = ev[r]
                    row = v * lanes + r
                    for c in range(dh // lanes):
                        sl = pl.ds(c * lanes, lanes)
                        hbufs[b][row, sl] = hbufs[b][row, sl] * er
                return carry
            lax.fori_loop(0, bsz // lanes, sub, 0)

        def fire_gather(p, b, j):
            pltpu.async_copy(h_hbm.at[dsts[p].at[pl.ds(j * bsz, bsz)]],
                             hbufs[b], gsems[b])

        def fire_scatter(p, b, j):
            pltpu.async_copy(hbufs[b],
                             u_sh.at[srcs[p].at[pl.ds(j * bsz, bsz)]],
                             ssems[b], add=True)

        def wait_gather(p, b, j):
            pltpu.make_async_copy(h_hbm.at[dsts[p].at[pl.ds(j * bsz, bsz)]],
                                  hbufs[b], gsems[b]).wait()

        def wait_scatter(p, b, j):
            pltpu.make_async_copy(
                hbufs[b], u_sh.at[srcs[p].at[pl.ds(j * bsz, bsz)]],
                ssems[b]).wait()

        look = 3
        def heavy_pass(p):
            for i in range(look):
                scalar_row(p, i)
                fire_gather(p, i, i)

            def ring(k, carry):
                for b in range(nbuf):
                    j = k * nbuf + b
                    bn = (b + look) % nbuf
                    if b < nbuf - look:
                        @pl.when(k > 0)
                        def _():
                            wait_scatter(p, bn, j + look - nbuf)
                        scalar_row(p, j + look)
                        fire_gather(p, bn, j + look)
                    else:
                        wait_scatter(p, bn, j + look - nbuf)

                        @pl.when(k < seg // nbuf - 1)
                        def _():
                            scalar_row(p, j + look)
                            fire_gather(p, bn, j + look)
                    wait_gather(p, b, j)
                    scale(p, b, j)
                    fire_scatter(p, b, j)
                return carry

            lax.fori_loop(0, seg // nbuf, ring, 0)
            for b in range(look, nbuf):
                wait_scatter(p, b, seg - nbuf + b)

        fire_stage(0, 0)

        def segpair(k, carry):
            g0 = 2 * k
            fire_stage(1, g0 + 1)
            wait_stage(0, g0)
            heavy_pass(0)

            @pl.when(k < nseg // 2 - 1)
            def _():
                fire_stage(0, g0 + 2)

            wait_stage(1, g0 + 1)
            heavy_pass(1)
            return carry

        lax.fori_loop(0, nseg // 2, segpair, 0)
        plsc.subcore_barrier()

        for k in range(rt // zr):
            pltpu.sync_copy(u_sh.at[pl.ds(row0 + k * zr, zr)],
                            u_out.at[cid, pl.ds(row0 + k * zr, zr)])

        @pl.when(cid == 0)
        def _():
            pltpu.sync_copy(s_v, s_out.at[pl.ds(sid * n, n)])

    return sc


def _tc_post_body(u_ref, s_ref, bias_ref, o_ref):
    n = o_ref.shape[2]
    acc = jnp.concatenate([u_ref[0, :n], u_ref[1, :n]], axis=1)
    s = jnp.sum(s_ref[...], axis=0)
    den = jnp.where(s > 0, s, 1.0)
    r = acc / den[:, None] + bias_ref[...][None, :]
    r = jnp.where(r > 0, r, jnp.exp(jnp.minimum(r, 0.0)) - 1.0)
    o_ref[...] = jnp.transpose(r)[None]


@functools.lru_cache(maxsize=None)
def _tc_post(n, d):
    return pl.pallas_call(
        _tc_post_body,
        out_shape=jax.ShapeDtypeStruct((1, d, n), jnp.float32),
    )


def kernel(x, edge_index, W, w1, b1, w2, b2, bias):
    _, d_in, n = x.shape
    d_out = W.shape[0]
    e = edge_index.shape[1]
    bsum = jnp.reshape(b1 + b2, (1, 1))
    h2, a1, a2 = _tc_pre(n, d_in, d_out)(x, W, w1, w2, bsum)
    u, s = _sc_main(n, e, d_out)(edge_index[0], edge_index[1], a1, a2, h2)
    return _tc_post(n, d_out)(u, jnp.reshape(s, (-1, n)), bias)
